# baseline (speedup 1.0000x reference)
import functools

import jax
import jax.numpy as jnp
from jax.experimental import pallas as pl
from jax.experimental.pallas import tpu as pltpu

_B = 2
_S = 2048
_D = 768
_H = 1024
_RH = 128
_E = 8

_TS = 1024
_NS = _S // _TS


def _nt_dot(a, b):
    return jax.lax.dot_general(
        a, b, (((1,), (1,)), ((), ())), preferred_element_type=jnp.float32
    )


def _w_copies(w1_hbm, w2_hbm, w3_hbm, w1s, w2s, w3s, sems, idx_s, b):
    cps = []
    for slot in range(2):
        e = idx_s[b, slot]
        cps.append(pltpu.make_async_copy(
            w1_hbm.at[e], w1s.at[slot], sems.at[slot, 0]))
        cps.append(pltpu.make_async_copy(
            w2_hbm.at[e], w2s.at[slot], sems.at[slot, 1]))
        cps.append(pltpu.make_async_copy(
            w3_hbm.at[e], w3s.at[slot], sems.at[slot, 2]))
    return cps


def _fused_kernel(x_hbm, rW1_ref, rb1_ref, rW2_ref, rb2_ref,
                  w1_hbm, w2_hbm, w3_hbm, b1_ref, b2_ref, b3_ref,
                  o_ref,
                  xs, w1s, w2s, w3s, sems, xsem, idx_s, gw_s):
    t = pl.program_id(0)

    @pl.when(t == 0)
    def _router_phase():
        cp = pltpu.make_async_copy(x_hbm, xs, xsem)
        cp.start()
        cp.wait()
        pooled = jnp.concatenate(
            [jnp.sum(xs[0], axis=0, keepdims=True),
             jnp.sum(xs[1], axis=0, keepdims=True)], axis=0) * (1.0 / _S)
        rh = jnp.maximum(_nt_dot(pooled, rW1_ref[...]) + rb1_ref[...], 0.0)
        logits = _nt_dot(rh, rW2_ref[...]) + rb2_ref[...]
        m = jnp.max(logits, axis=1, keepdims=True)
        eg = jnp.exp(logits - m)
        gate = eg / jnp.sum(eg, axis=1, keepdims=True)
        eiota = jax.lax.broadcasted_iota(jnp.int32, gate.shape, 1)
        v1 = jnp.max(gate, axis=1, keepdims=True)
        i1 = jnp.min(jnp.where(gate >= v1, eiota, _E), axis=1, keepdims=True)
        masked = jnp.where(eiota == i1, -jnp.float32(jnp.inf), gate)
        v2 = jnp.max(masked, axis=1, keepdims=True)
        i2 = jnp.min(jnp.where(masked >= v2, eiota, _E), axis=1, keepdims=True)
        idx_s[...] = jnp.concatenate([i1, i2], axis=1).astype(jnp.int32)
        gw_s[...] = jnp.concatenate([v1, v2], axis=1)
        for cp in _w_copies(w1_hbm, w2_hbm, w3_hbm, w1s, w2s, w3s,
                            sems, idx_s, 0):
            cp.start()

    @pl.when(t > 0)
    def _expert_phase():
        u = t - 1
        b = u // _NS
        s = u % _NS

        @pl.when((s == 0) & (b > 0))
        def _start_row():
            for cp in _w_copies(w1_hbm, w2_hbm, w3_hbm, w1s, w2s, w3s,
                                sems, idx_s, b):
                cp.start()

        @pl.when(s == 0)
        def _await_weights():
            for cp in _w_copies(w1_hbm, w2_hbm, w3_hbm, w1s, w2s, w3s,
                                sems, idx_s, b):
                cp.wait()

        start = pl.multiple_of(s * _TS, _TS)
        xb = xs[b, pl.ds(start, _TS), :]

        def expert_p(slot):
            e = idx_s[b, slot]
            h1 = jnp.maximum(_nt_dot(xb, w1s[slot]) + b1_ref[e], 0.0)
            h2 = jnp.maximum(_nt_dot(h1, w2s[slot]) + b2_ref[e], 0.0)
            out = _nt_dot(h2, w3s[slot]) + b3_ref[e]
            mx = jnp.max(out, axis=1, keepdims=True)
            eo = jnp.exp(out - mx)
            return eo / jnp.sum(eo, axis=1, keepdims=True)

        gwb = gw_s[pl.ds(b, 1), :]
        o_ref[0] = gwb[0, 0] * expert_p(0)
        o_ref[0] += gwb[0, 1] * expert_p(1)


def kernel(x, rW1, rb1, rW2, rb2, eW1, eb1, eW2, eb2, eW3, eb3):
    def omap(t):
        u = jnp.maximum(t - 1, 0)
        return (u // _NS, u % _NS, 0)

    return pl.pallas_call(
        _fused_kernel,
        grid=(1 + _B * _NS,),
        in_specs=[
            pl.BlockSpec(memory_space=pl.ANY),
            pl.BlockSpec((_RH, _D), lambda t: (0, 0)),
            pl.BlockSpec((1, _RH), lambda t: (0, 0)),
            pl.BlockSpec((_E, _RH), lambda t: (0, 0)),
            pl.BlockSpec((1, _E), lambda t: (0, 0)),
            pl.BlockSpec(memory_space=pl.ANY),
            pl.BlockSpec(memory_space=pl.ANY),
            pl.BlockSpec(memory_space=pl.ANY),
            pl.BlockSpec((_E, 1, _H), lambda t: (0, 0, 0)),
            pl.BlockSpec((_E, 1, _H), lambda t: (0, 0, 0)),
            pl.BlockSpec((_E, 1, _D), lambda t: (0, 0, 0)),
        ],
        out_specs=pl.BlockSpec((1, _TS, _D), omap),
        out_shape=jax.ShapeDtypeStruct((_B, _S, _D), jnp.float32),
        scratch_shapes=[
            pltpu.VMEM((_B, _S, _D), jnp.float32),
            pltpu.VMEM((2, _H, _D), jnp.float32),
            pltpu.VMEM((2, _H, _H), jnp.float32),
            pltpu.VMEM((2, _D, _H), jnp.float32),
            pltpu.SemaphoreType.DMA((2, 3)),
            pltpu.SemaphoreType.DMA,
            pltpu.VMEM((_B, 2), jnp.int32),
            pltpu.VMEM((_B, 2), jnp.float32),
        ],
    )(x, rW1, rb1.reshape(1, _RH), rW2, rb2.reshape(1, _E),
      eW1, eW2, eW3,
      eb1.reshape(_E, 1, _H), eb2.reshape(_E, 1, _H), eb3.reshape(_E, 1, _D))


# speedup vs baseline: 1.0952x; 1.0952x over previous
import functools

import jax
import jax.numpy as jnp
from jax.experimental import pallas as pl
from jax.experimental.pallas import tpu as pltpu

_B = 2
_S = 2048
_D = 768
_H = 1024
_RH = 128
_E = 8

_TS = 512
_NS = _S // _TS


def _nt_dot(a, b):
    return jax.lax.dot_general(
        a, b, (((1,), (1,)), ((), ())), preferred_element_type=jnp.float32
    )


def _w_copies(w1_hbm, w2_hbm, w3_hbm, w1s, w2s, w3s, sems, idx_s, b):
    cps = []
    for slot in range(2):
        e = idx_s[b, slot]
        g = 2 * b + slot
        cps.append(pltpu.make_async_copy(
            w1_hbm.at[e], w1s.at[g], sems.at[g, 0]))
        cps.append(pltpu.make_async_copy(
            w2_hbm.at[e], w2s.at[g], sems.at[g, 1]))
        cps.append(pltpu.make_async_copy(
            w3_hbm.at[e], w3s.at[g], sems.at[g, 2]))
    return cps


def _fused_kernel(x_hbm, rW1_ref, rb1_ref, rW2_ref, rb2_ref,
                  w1_hbm, w2_hbm, w3_hbm, b1_ref, b2_ref, b3_ref,
                  o_ref,
                  xs, w1s, w2s, w3s, sems, xsem, idx_s, gw_s):
    t = pl.program_id(0)

    @pl.when(t == 0)
    def _router_phase():
        cp = pltpu.make_async_copy(x_hbm, xs, xsem)
        cp.start()
        cp.wait()
        pooled = jnp.concatenate(
            [jnp.sum(xs[0], axis=0, keepdims=True),
             jnp.sum(xs[1], axis=0, keepdims=True)], axis=0) * (1.0 / _S)
        rh = jnp.maximum(_nt_dot(pooled, rW1_ref[...]) + rb1_ref[...], 0.0)
        logits = _nt_dot(rh, rW2_ref[...]) + rb2_ref[...]
        m = jnp.max(logits, axis=1, keepdims=True)
        eg = jnp.exp(logits - m)
        gate = eg / jnp.sum(eg, axis=1, keepdims=True)
        eiota = jax.lax.broadcasted_iota(jnp.int32, gate.shape, 1)
        v1 = jnp.max(gate, axis=1, keepdims=True)
        i1 = jnp.min(jnp.where(gate >= v1, eiota, _E), axis=1, keepdims=True)
        masked = jnp.where(eiota == i1, -jnp.float32(jnp.inf), gate)
        v2 = jnp.max(masked, axis=1, keepdims=True)
        i2 = jnp.min(jnp.where(masked >= v2, eiota, _E), axis=1, keepdims=True)
        idx_s[...] = jnp.concatenate([i1, i2], axis=1).astype(jnp.int32)
        gw_s[...] = jnp.concatenate([v1, v2], axis=1)
        for cp in _w_copies(w1_hbm, w2_hbm, w3_hbm, w1s, w2s, w3s,
                            sems, idx_s, 0):
            cp.start()

    @pl.when(t > 0)
    def _expert_phase():
        u = t - 1
        b = u // _NS
        s = u % _NS

        @pl.when(s == 0)
        def _await_weights():
            for cp in _w_copies(w1_hbm, w2_hbm, w3_hbm, w1s, w2s, w3s,
                                sems, idx_s, b):
                cp.wait()

        @pl.when((s == 0) & (b + 1 < _B))
        def _prefetch_next_row():
            for cp in _w_copies(w1_hbm, w2_hbm, w3_hbm, w1s, w2s, w3s,
                                sems, idx_s, b + 1):
                cp.start()

        start = pl.multiple_of(s * _TS, _TS)
        xb = xs[b, pl.ds(start, _TS), :]

        def expert_p(slot):
            e = idx_s[b, slot]
            g = 2 * b + slot
            h1 = jnp.maximum(_nt_dot(xb, w1s[g]) + b1_ref[e], 0.0)
            h2 = jnp.maximum(_nt_dot(h1, w2s[g]) + b2_ref[e], 0.0)
            out = _nt_dot(h2, w3s[g]) + b3_ref[e]
            mx = jnp.max(out, axis=1, keepdims=True)
            eo = jnp.exp(out - mx)
            return eo / jnp.sum(eo, axis=1, keepdims=True)

        gwb = gw_s[pl.ds(b, 1), :]
        o_ref[0] = gwb[0, 0] * expert_p(0)
        o_ref[0] += gwb[0, 1] * expert_p(1)


def kernel(x, rW1, rb1, rW2, rb2, eW1, eb1, eW2, eb2, eW3, eb3):
    def omap(t):
        u = jnp.maximum(t - 1, 0)
        return (u // _NS, u % _NS, 0)

    return pl.pallas_call(
        _fused_kernel,
        grid=(1 + _B * _NS,),
        in_specs=[
            pl.BlockSpec(memory_space=pl.ANY),
            pl.BlockSpec((_RH, _D), lambda t: (0, 0)),
            pl.BlockSpec((1, _RH), lambda t: (0, 0)),
            pl.BlockSpec((_E, _RH), lambda t: (0, 0)),
            pl.BlockSpec((1, _E), lambda t: (0, 0)),
            pl.BlockSpec(memory_space=pl.ANY),
            pl.BlockSpec(memory_space=pl.ANY),
            pl.BlockSpec(memory_space=pl.ANY),
            pl.BlockSpec((_E, 1, _H), lambda t: (0, 0, 0)),
            pl.BlockSpec((_E, 1, _H), lambda t: (0, 0, 0)),
            pl.BlockSpec((_E, 1, _D), lambda t: (0, 0, 0)),
        ],
        out_specs=pl.BlockSpec((1, _TS, _D), omap),
        out_shape=jax.ShapeDtypeStruct((_B, _S, _D), jnp.float32),
        compiler_params=pltpu.CompilerParams(
            vmem_limit_bytes=100 * 1024 * 1024),
        scratch_shapes=[
            pltpu.VMEM((_B, _S, _D), jnp.float32),
            pltpu.VMEM((2 * _B, _H, _D), jnp.float32),
            pltpu.VMEM((2 * _B, _H, _H), jnp.float32),
            pltpu.VMEM((2 * _B, _D, _H), jnp.float32),
            pltpu.SemaphoreType.DMA((2 * _B, 3)),
            pltpu.SemaphoreType.DMA,
            pltpu.VMEM((_B, 2), jnp.int32),
            pltpu.VMEM((_B, 2), jnp.float32),
        ],
    )(x, rW1, rb1.reshape(1, _RH), rW2, rb2.reshape(1, _E),
      eW1, eW2, eW3,
      eb1.reshape(_E, 1, _H), eb2.reshape(_E, 1, _H), eb3.reshape(_E, 1, _D))
